# baseline (speedup 1.0000x reference)
import functools

import jax
import jax.numpy as jnp
from jax import lax
from jax.experimental import pallas as pl
from jax.experimental.pallas import tpu as pltpu
from jax.experimental.pallas import tpu_sc as plsc

C = 384
CR = 96
KNN = 8
REP = C // CR
HB = 32

NWORK = 32
LANES = 16
FCH = CR // LANES
CPAD = 128


def _reduce_body(x_ref, wf_ref, wgcn_ref, out_ref):
    xb = x_ref[0]
    W = xb.shape[1]
    s = jnp.sum(xb, axis=1) * (1.0 / W)
    sw = s * wf_ref[...]
    c_iota = lax.broadcasted_iota(jnp.int32, (C, CR), 0)
    g_iota = lax.broadcasted_iota(jnp.int32, (C, CR), 1)
    P = jnp.where(c_iota // REP == g_iota, 1.0, 0.0)
    xr = jnp.dot(sw, P, preferred_element_type=jnp.float32)
    sup = jnp.dot(xr, wgcn_ref[...],
                  preferred_element_type=jnp.float32)
    out_ref[0] = jnp.concatenate(
        [sup, jnp.zeros((sup.shape[0], CPAD - CR), jnp.float32)], axis=1)


def _graph_idx_body(ukt_ref, cols_ref, wn_ref):
    U = ukt_ref[0]
    H = U.shape[1]
    ss = jnp.sum(U * U, axis=0, keepdims=True)
    inv = 1.0 / jnp.maximum(jnp.sqrt(ss), 1e-12)
    Un = U * inv
    sim = lax.dot_general(Un, Un, (((0,), (0,)), ((), ())),
                          preferred_element_type=jnp.float32)

    col = lax.broadcasted_iota(jnp.int32, (H, H), 1)
    work = sim
    deg = jnp.zeros((H, 1), jnp.float32)
    jstars = []
    vals = []
    for _ in range(KNN):
        m = jnp.max(work, axis=1, keepdims=True)
        cand = jnp.where(work == m, col, jnp.int32(1 << 30))
        jstar = jnp.min(cand, axis=1, keepdims=True)
        onehot = col == jstar
        jstars.append(jstar)
        vals.append(m)
        deg = deg + m
        work = jnp.where(onehot, jnp.float32(-1e30), work)

    dis = lax.rsqrt(deg)
    disrow = jnp.transpose(dis)
    wns = []
    for r in range(KNN):
        discol = jnp.sum(jnp.where(col == jstars[r], disrow, 0.0),
                         axis=1, keepdims=True)
        wns.append(vals[r] * dis * discol)
    cols = jnp.concatenate(jstars, axis=1)
    wn = jnp.concatenate(wns, axis=1)
    b = pl.program_id(0)
    cols_ref[0] = cols + b * H
    wn_ref[0] = jnp.broadcast_to(wn[:, :, None], (H, KNN, LANES))


def _make_sc_msg():
    NV = 4 * 224
    NPW = NV // NWORK
    EPW = NPW * KNN
    EH = EPW // 2
    mesh = plsc.VectorSubcoreMesh(core_axis_name="c", subcore_axis_name="s")

    @functools.partial(
        pl.kernel,
        mesh=mesh,
        out_type=jax.ShapeDtypeStruct((NV * CR,), jnp.float32),
        scratch_types=[
            pltpu.VMEM((EH,), jnp.int32),
            pltpu.VMEM((EH,), jnp.int32),
            pltpu.VMEM((EH, CPAD), jnp.float32),
            pltpu.VMEM((EH, CPAD), jnp.float32),
            pltpu.VMEM((EPW * LANES,), jnp.float32),
            pltpu.VMEM((NPW * CR,), jnp.float32),
            pltpu.SemaphoreType.DMA,
        ],
    )
    def sc_msg(sup_hbm, gcols_hbm, wnb_hbm, out_hbm,
               idx_a, idx_b, rows_a, rows_b, wn_v, out_v, sem):
        w = lax.axis_index("s") * 2 + lax.axis_index("c")
        ebase = w * EPW
        pltpu.sync_copy(gcols_hbm.at[pl.ds(ebase, EH)], idx_a)
        pltpu.sync_copy(gcols_hbm.at[pl.ds(ebase + EH, EH)], idx_b)
        pltpu.async_copy(sup_hbm.at[idx_a], rows_a, sem).wait()
        pltpu.async_copy(sup_hbm.at[idx_b], rows_b, sem).wait()
        pltpu.sync_copy(wnb_hbm.at[pl.ds(ebase * LANES, EPW * LANES)], wn_v)
        for n in range(NPW):
            acc = [jnp.zeros((LANES,), jnp.float32) for _ in range(FCH)]
            for k in range(KNN):
                e = n * KNN + k
                wv = wn_v[pl.ds(e * LANES, LANES)]
                rv = rows_a if e < EH else rows_b
                er = e if e < EH else e - EH
                for j in range(FCH):
                    acc[j] = acc[j] + wv * rv[er, pl.ds(j * LANES, LANES)]
            for j in range(FCH):
                out_v[pl.ds(n * CR + j * LANES, LANES)] = acc[j]
        pltpu.sync_copy(out_v, out_hbm.at[pl.ds(w * NPW * CR, NPW * CR)])

    return sc_msg


_sc_msg = _make_sc_msg()


def _expand_body(on_ref, brow_ref, we_ref, out_ref):
    on = jnp.maximum(on_ref[0] + brow_ref[...], 0.0)
    Wdim = out_ref.shape[2]
    g_iota = lax.broadcasted_iota(jnp.int32, (CR, C), 0)
    c_iota = lax.broadcasted_iota(jnp.int32, (CR, C), 1)
    E = jnp.where(c_iota // REP == g_iota, 1.0, 0.0) * we_ref[...]
    scale = jnp.dot(on, E, preferred_element_type=jnp.float32)
    out_ref[0] = jnp.broadcast_to(scale[:, None, :], (on.shape[0], Wdim, C))


@jax.jit
def kernel(x, Uk, W_reduce, W_gcn, b_gcn, W_expand):
    B, Cc, H, W = x.shape
    NHB = H // HB
    K = Uk.shape[-1]

    xt = jnp.transpose(x, (0, 2, 3, 1))
    ukt = jnp.transpose(Uk, (0, 2, 1))
    wf = W_reduce.reshape(1, Cc)
    we = W_expand.reshape(1, Cc)
    br = b_gcn.reshape(1, CR)

    support = pl.pallas_call(
        _reduce_body,
        grid=(B, NHB),
        in_specs=[
            pl.BlockSpec((1, HB, W, Cc), lambda b, h: (b, h, 0, 0)),
            pl.BlockSpec((1, Cc), lambda b, h: (0, 0)),
            pl.BlockSpec((CR, CR), lambda b, h: (0, 0)),
        ],
        out_specs=pl.BlockSpec((1, HB, CPAD), lambda b, h: (b, h, 0)),
        out_shape=jax.ShapeDtypeStruct((B, H, CPAD), jnp.float32),
    )(xt, wf, W_gcn)

    gcols, wnb = pl.pallas_call(
        _graph_idx_body,
        grid=(B,),
        in_specs=[
            pl.BlockSpec((1, K, H), lambda b: (b, 0, 0)),
        ],
        out_specs=[
            pl.BlockSpec((1, H, KNN), lambda b: (b, 0, 0)),
            pl.BlockSpec((1, H, KNN, LANES), lambda b: (b, 0, 0, 0)),
        ],
        out_shape=[
            jax.ShapeDtypeStruct((B, H, KNN), jnp.int32),
            jax.ShapeDtypeStruct((B, H, KNN, LANES), jnp.float32),
        ],
    )(ukt)

    msg_flat = _sc_msg(
        support.reshape(B * H, CPAD),
        gcols.reshape(B * H * KNN),
        wnb.reshape(B * H * KNN * LANES),
    )
    msg = msg_flat.reshape(B, H, CR)

    yt = pl.pallas_call(
        _expand_body,
        grid=(B, NHB),
        in_specs=[
            pl.BlockSpec((1, HB, CR), lambda b, h: (b, h, 0)),
            pl.BlockSpec((1, CR), lambda b, h: (0, 0)),
            pl.BlockSpec((1, Cc), lambda b, h: (0, 0)),
        ],
        out_specs=pl.BlockSpec((1, HB, W, Cc), lambda b, h: (b, h, 0, 0)),
        out_shape=jax.ShapeDtypeStruct((B, H, W, Cc), jnp.float32),
    )(msg, br, we)
    return jnp.transpose(yt, (0, 3, 1, 2))


# speedup vs baseline: 1.0211x; 1.0211x over previous
import functools

import jax
import jax.numpy as jnp
from jax import lax
from jax.experimental import pallas as pl
from jax.experimental.pallas import tpu as pltpu
from jax.experimental.pallas import tpu_sc as plsc

C = 384
CR = 96
KNN = 8
REP = C // CR
HB = 32

NWORK = 32
LANES = 16
FCH = CR // LANES
CPAD = 128


def _reduce_body(x_ref, wf_ref, wgcn_ref, ukt_ref, out_ref, cols_ref, wn_ref):
    xb = x_ref[0]
    W = xb.shape[1]
    s = jnp.sum(xb, axis=1) * (1.0 / W)
    sw = s * wf_ref[...]
    c_iota = lax.broadcasted_iota(jnp.int32, (C, CR), 0)
    g_iota = lax.broadcasted_iota(jnp.int32, (C, CR), 1)
    P = jnp.where(c_iota // REP == g_iota, 1.0, 0.0)
    xr = jnp.dot(sw, P, preferred_element_type=jnp.float32)
    sup = jnp.dot(xr, wgcn_ref[...],
                  preferred_element_type=jnp.float32)
    out_ref[0] = jnp.concatenate(
        [sup, jnp.zeros((sup.shape[0], CPAD - CR), jnp.float32)], axis=1)

    @pl.when(pl.program_id(1) == 0)
    def _():
        _graph_idx_body(ukt_ref, cols_ref, wn_ref)


def _graph_idx_body(ukt_ref, cols_ref, wn_ref):
    U = ukt_ref[0]
    H = U.shape[1]
    ss = jnp.sum(U * U, axis=0, keepdims=True)
    inv = 1.0 / jnp.maximum(jnp.sqrt(ss), 1e-12)
    Un = U * inv
    sim = lax.dot_general(Un, Un, (((0,), (0,)), ((), ())),
                          preferred_element_type=jnp.float32)

    col = lax.broadcasted_iota(jnp.int32, (H, H), 1)
    work = sim
    deg = jnp.zeros((H, 1), jnp.float32)
    jstars = []
    vals = []
    for _ in range(KNN):
        m = jnp.max(work, axis=1, keepdims=True)
        cand = jnp.where(work == m, col, jnp.int32(1 << 30))
        jstar = jnp.min(cand, axis=1, keepdims=True)
        onehot = col == jstar
        jstars.append(jstar)
        vals.append(m)
        deg = deg + m
        work = jnp.where(onehot, jnp.float32(-1e30), work)

    dis = lax.rsqrt(deg)
    disrow = jnp.transpose(dis)
    wns = []
    for r in range(KNN):
        discol = jnp.sum(jnp.where(col == jstars[r], disrow, 0.0),
                         axis=1, keepdims=True)
        wns.append(vals[r] * dis * discol)
    cols = jnp.concatenate(jstars, axis=1)
    wn = jnp.concatenate(wns, axis=1)
    b = pl.program_id(0)
    cols_ref[0] = cols + b * H
    wn_ref[0] = jnp.broadcast_to(wn[:, :, None], (H, KNN, LANES))


def _make_sc_msg():
    NV = 4 * 224
    NPW = NV // NWORK
    EPW = NPW * KNN
    EH = EPW // 2
    mesh = plsc.VectorSubcoreMesh(core_axis_name="c", subcore_axis_name="s")

    @functools.partial(
        pl.kernel,
        mesh=mesh,
        out_type=jax.ShapeDtypeStruct((NV * CR,), jnp.float32),
        scratch_types=[
            pltpu.VMEM((EH,), jnp.int32),
            pltpu.VMEM((EH,), jnp.int32),
            pltpu.VMEM((EH, CPAD), jnp.float32),
            pltpu.VMEM((EH, CPAD), jnp.float32),
            pltpu.VMEM((EPW * LANES,), jnp.float32),
            pltpu.VMEM((NPW * CR,), jnp.float32),
            pltpu.SemaphoreType.DMA,
        ],
    )
    def sc_msg(sup_hbm, gcols_hbm, wnb_hbm, out_hbm,
               idx_a, idx_b, rows_a, rows_b, wn_v, out_v, sem):
        w = lax.axis_index("s") * 2 + lax.axis_index("c")
        ebase = w * EPW
        pltpu.sync_copy(gcols_hbm.at[pl.ds(ebase, EH)], idx_a)
        pltpu.sync_copy(gcols_hbm.at[pl.ds(ebase + EH, EH)], idx_b)
        pltpu.async_copy(sup_hbm.at[idx_a], rows_a, sem).wait()
        pltpu.async_copy(sup_hbm.at[idx_b], rows_b, sem).wait()
        pltpu.sync_copy(wnb_hbm.at[pl.ds(ebase * LANES, EPW * LANES)], wn_v)
        for n in range(NPW):
            acc = [jnp.zeros((LANES,), jnp.float32) for _ in range(FCH)]
            for k in range(KNN):
                e = n * KNN + k
                wv = wn_v[pl.ds(e * LANES, LANES)]
                rv = rows_a if e < EH else rows_b
                er = e if e < EH else e - EH
                for j in range(FCH):
                    acc[j] = acc[j] + wv * rv[er, pl.ds(j * LANES, LANES)]
            for j in range(FCH):
                out_v[pl.ds(n * CR + j * LANES, LANES)] = acc[j]
        pltpu.sync_copy(out_v, out_hbm.at[pl.ds(w * NPW * CR, NPW * CR)])

    return sc_msg


_sc_msg = _make_sc_msg()


def _expand_body(on_ref, brow_ref, we_ref, out_ref):
    on = jnp.maximum(on_ref[0] + brow_ref[...], 0.0)
    Wdim = out_ref.shape[2]
    g_iota = lax.broadcasted_iota(jnp.int32, (CR, C), 0)
    c_iota = lax.broadcasted_iota(jnp.int32, (CR, C), 1)
    E = jnp.where(c_iota // REP == g_iota, 1.0, 0.0) * we_ref[...]
    scale = jnp.dot(on, E, preferred_element_type=jnp.float32)
    out_ref[0] = jnp.broadcast_to(scale[:, None, :], (on.shape[0], Wdim, C))


@jax.jit
def kernel(x, Uk, W_reduce, W_gcn, b_gcn, W_expand):
    B, Cc, H, W = x.shape
    NHB = H // HB
    K = Uk.shape[-1]

    xt = jnp.transpose(x, (0, 2, 3, 1))
    ukt = jnp.transpose(Uk, (0, 2, 1))
    wf = W_reduce.reshape(1, Cc)
    we = W_expand.reshape(1, Cc)
    br = b_gcn.reshape(1, CR)

    support, gcols, wnb = pl.pallas_call(
        _reduce_body,
        grid=(B, NHB),
        in_specs=[
            pl.BlockSpec((1, HB, W, Cc), lambda b, h: (b, h, 0, 0)),
            pl.BlockSpec((1, Cc), lambda b, h: (0, 0)),
            pl.BlockSpec((CR, CR), lambda b, h: (0, 0)),
            pl.BlockSpec((1, K, H), lambda b, h: (b, 0, 0)),
        ],
        out_specs=[
            pl.BlockSpec((1, HB, CPAD), lambda b, h: (b, h, 0)),
            pl.BlockSpec((1, H, KNN), lambda b, h: (b, 0, 0)),
            pl.BlockSpec((1, H, KNN, LANES), lambda b, h: (b, 0, 0, 0)),
        ],
        out_shape=[
            jax.ShapeDtypeStruct((B, H, CPAD), jnp.float32),
            jax.ShapeDtypeStruct((B, H, KNN), jnp.int32),
            jax.ShapeDtypeStruct((B, H, KNN, LANES), jnp.float32),
        ],
    )(xt, wf, W_gcn, ukt)

    msg_flat = _sc_msg(
        support.reshape(B * H, CPAD),
        gcols.reshape(B * H * KNN),
        wnb.reshape(B * H * KNN * LANES),
    )
    msg = msg_flat.reshape(B, H, CR)

    yt = pl.pallas_call(
        _expand_body,
        grid=(B, NHB),
        in_specs=[
            pl.BlockSpec((1, HB, CR), lambda b, h: (b, h, 0)),
            pl.BlockSpec((1, CR), lambda b, h: (0, 0)),
            pl.BlockSpec((1, Cc), lambda b, h: (0, 0)),
        ],
        out_specs=pl.BlockSpec((1, HB, W, Cc), lambda b, h: (b, h, 0, 0)),
        out_shape=jax.ShapeDtypeStruct((B, H, W, Cc), jnp.float32),
    )(msg, br, we)
    return jnp.transpose(yt, (0, 3, 1, 2))
